# baseline (speedup 1.0000x reference)
import jax
import jax.numpy as jnp
from jax import lax
from jax.experimental import pallas as pl
from jax.experimental.pallas import tpu as pltpu
from jax.experimental.pallas import tpu_sc as plsc

N = 10000
E = 320000
DIM = 128
HDIM = 64
H = 8
HH = 4
HD = 16

C = 128
NCHUNKS = E // C
NCORES = 2
NSUB = 16
CPT = (NCHUNKS + NSUB - 1) // NSUB

_MESH = plsc.VectorSubcoreMesh(
    core_axis_name="c", subcore_axis_name="s",
    num_cores=NCORES, num_subcores=NSUB)

_SC_PARAMS = pltpu.CompilerParams(
    use_tc_tiling_on_sc=False, needs_layout_passes=False)


_PROJ_BLK = 1000


def _proj_body(x_ref, wq, bq, wk, bk, wv, bv, q_out, k_out, v_out):
    xb = x_ref[...]
    dn = (((1,), (1,)), ((), ()))
    q = lax.dot_general(xb, wq[...], dn,
                        preferred_element_type=jnp.float32) + bq[...]
    k = lax.dot_general(xb, wk[...], dn,
                        preferred_element_type=jnp.float32) + bk[...]
    v = lax.dot_general(xb, wv[...], dn,
                        preferred_element_type=jnp.float32) + bv[...]
    q_out[...] = jnp.stack([q[:, :HDIM], q[:, HDIM:]])
    k_out[...] = jnp.stack([k[:, :HDIM], k[:, HDIM:]])
    v_out[...] = jnp.stack([v[:, :HDIM], v[:, HDIM:]])


def _project(x, Wq, bq, Wk, bk, Wv, bv):
    full = pl.BlockSpec((DIM, DIM), lambda i: (0, 0))
    brow = pl.BlockSpec((1, DIM), lambda i: (0, 0))
    blk = pl.BlockSpec((_PROJ_BLK, DIM), lambda i: (i, 0))
    sblk = pl.BlockSpec((NCORES, _PROJ_BLK, HDIM), lambda i: (0, i, 0))
    ssds = jax.ShapeDtypeStruct((NCORES, N, HDIM), jnp.float32)
    return pl.pallas_call(
        _proj_body,
        grid=(N // _PROJ_BLK,),
        in_specs=[blk, full, brow, full, brow, full, brow],
        out_specs=[sblk] * 3,
        out_shape=[ssds] * 3,
    )(x, Wq, bq.reshape(1, DIM), Wk, bk.reshape(1, DIM), Wv, bv.reshape(1, DIM))


def _scores_body(ktab_hbm, qtab_hbm, ei_hbm, zero4_hbm,
                 score_hbm, sum_hbm,
                 ei_a, ei_b, kr_a, kr_b, qr_a, qr_b, scores_v, sum_acc,
                 sem):
    cid = lax.axis_index("c")
    sid = lax.axis_index("s")

    EI = (ei_a, ei_b)
    KR = (kr_a, kr_b)
    QR = (qr_a, qr_b)

    @pl.when(sid == 0)
    def _zero():
        pltpu.sync_copy(zero4_hbm, sum_acc)

    plsc.subcore_barrier()

    lanes = lax.iota(jnp.int32, HD)

    def load_idx(chunk, p):
        pltpu.sync_copy(ei_hbm.at[:, pl.ds(chunk * C, C)], EI[p])

    def issue_gathers(p):
        dk = pltpu.async_copy(ktab_hbm.at[cid].at[EI[p].at[0]], KR[p], sem)
        dq = pltpu.async_copy(qtab_hbm.at[cid].at[EI[p].at[1]], QR[p], sem)
        return dk, dq

    def compute(p):
        krows, qrows = KR[p], QR[p]

        def group_body(g, c2):
            e0 = g * HD
            rowv = jnp.full((HD,), e0, jnp.int32) + lanes
            for h in range(HH):
                acc = jnp.zeros((HD,), jnp.float32)
                for d in range(HD):
                    colv = jnp.full((HD,), h * HD + d, jnp.int32)
                    kv = plsc.load_gather(krows, [rowv, colv])
                    qv = plsc.load_gather(qrows, [rowv, colv])
                    acc = acc + kv * qv
                plsc.store_scatter(scores_v,
                                   [rowv, jnp.full((HD,), h, jnp.int32)],
                                   acc)
            return c2

        lax.fori_loop(0, C // HD, group_body, 0)

    def do_writes(chunk, p):
        base = chunk * C
        pltpu.sync_copy(scores_v, score_hbm.at[cid, pl.ds(base, C)])
        pltpu.sync_copy(scores_v, sum_acc.at[EI[p].at[1]], add=True)

    load_idx(sid, 0)
    dk0, dq0 = issue_gathers(0)
    dk0.wait()
    dq0.wait()

    def dbl_body(i2, carry):
        for p in (0, 1):
            i = i2 * 2 + p
            cur = i * NSUB + sid
            nxt = cur + NSUB

            @pl.when(nxt < NCHUNKS)
            def _steady(p=p, cur=cur, nxt=nxt):
                load_idx(nxt, 1 - p)
                dk, dq = issue_gathers(1 - p)
                compute(p)
                do_writes(cur, p)
                dk.wait()
                dq.wait()

            @pl.when((cur < NCHUNKS) & (nxt >= NCHUNKS))
            def _tail(p=p, cur=cur):
                compute(p)
                do_writes(cur, p)
        return carry

    lax.fori_loop(0, (CPT + 1) // 2, dbl_body, 0)

    plsc.subcore_barrier()

    @pl.when(sid == 0)
    def _out():
        pltpu.sync_copy(sum_acc, sum_hbm.at[cid])


_scores_call = pl.kernel(
    _scores_body,
    out_type=(jax.ShapeDtypeStruct((NCORES, E, HH), jnp.float32),
              jax.ShapeDtypeStruct((NCORES, N, HH), jnp.float32)),
    mesh=_MESH,
    compiler_params=_SC_PARAMS,
    scratch_types=[
        pltpu.VMEM((2, C), jnp.int32),
        pltpu.VMEM((2, C), jnp.int32),
        pltpu.VMEM((C, HDIM), jnp.float32),
        pltpu.VMEM((C, HDIM), jnp.float32),
        pltpu.VMEM((C, HDIM), jnp.float32),
        pltpu.VMEM((C, HDIM), jnp.float32),
        pltpu.VMEM((C, HH), jnp.float32),
        pltpu.VMEM_SHARED((N, HH), jnp.float32),
        pltpu.SemaphoreType.DMA,
    ],
)


def _inv_body(p_ref, inv_ref):
    inv_ref[...] = 1.0 / p_ref[...]


def _inv_sum(sums):
    return pl.pallas_call(
        _inv_body,
        out_shape=jax.ShapeDtypeStruct((NCORES, N, HH), jnp.float32),
    )(sums)


def _agg_body(vtab_hbm, ei_hbm, score_hbm, inv_hbm, zero64_hbm,
              outpart_hbm,
              ei_a, ei_b, vr_a, vr_b, sc_a, sc_b, inv_t, out_acc,
              sem):
    cid = lax.axis_index("c")
    sid = lax.axis_index("s")

    EI = (ei_a, ei_b)
    VR = (vr_a, vr_b)
    SCV = (sc_a, sc_b)

    @pl.when(sid == 0)
    def _zero():
        pltpu.sync_copy(zero64_hbm, out_acc)

    pltpu.sync_copy(inv_hbm.at[cid], inv_t)
    plsc.subcore_barrier()

    lanes = lax.iota(jnp.int32, HD)
    row_off = lanes >> 2
    col_idx = lanes & (HH - 1)

    def load_idx(chunk, p):
        pltpu.sync_copy(ei_hbm.at[:, pl.ds(chunk * C, C)], EI[p])

    def issue_gathers(chunk, p):
        dv = pltpu.async_copy(vtab_hbm.at[cid].at[EI[p].at[0]], VR[p], sem)
        dscore = pltpu.async_copy(score_hbm.at[cid, pl.ds(chunk * C, C)],
                                  SCV[p], sem)
        return dv, dscore

    def compute(p):
        vrows, scores_v = VR[p], SCV[p]
        ei = EI[p]

        def group_body(g, c2):
            dvec = ei[1, pl.ds(g * HD, HD)]
            for jq in range(HD // 4):
                e0 = g * HD + 4 * jq
                rowi = jnp.zeros((HD,), jnp.int32)
                for ej in range(4):
                    mask = (lanes >= ej * HH) & (lanes < (ej + 1) * HH)
                    rowi = jnp.where(
                        mask,
                        jnp.full((HD,), dvec[4 * jq + ej], jnp.int32),
                        rowi)
                rowv = jnp.full((HD,), e0, jnp.int32) + row_off
                sv = plsc.load_gather(scores_v, [rowv, col_idx])
                iv = plsc.load_gather(inv_t, [rowi, col_idx])
                w = sv * iv
                for ej in range(4):
                    for h in range(HH):
                        ws = jnp.full((HD,), w[ej * HH + h], jnp.float32)
                        vrows[e0 + ej, pl.ds(h * HD, HD)] = (
                            vrows[e0 + ej, pl.ds(h * HD, HD)] * ws)
            return c2

        lax.fori_loop(0, C // HD, group_body, 0)

    def do_writes(p):
        pltpu.sync_copy(VR[p], out_acc.at[EI[p].at[1]], add=True)

    def chunk_body(i, carry):
        cur = i * NSUB + sid

        @pl.when(cur < NCHUNKS)
        def _():
            load_idx(cur, 0)
            dv, dscore = issue_gathers(cur, 0)
            dv.wait()
            dscore.wait()
            compute(0)
            do_writes(0)

        return carry

    lax.fori_loop(0, CPT, chunk_body, 0)

    plsc.subcore_barrier()

    @pl.when(sid == 0)
    def _out():
        pltpu.sync_copy(out_acc, outpart_hbm.at[cid])


_agg_call = pl.kernel(
    _agg_body,
    out_type=jax.ShapeDtypeStruct((NCORES, N, HDIM), jnp.float32),
    mesh=_MESH,
    compiler_params=_SC_PARAMS,
    scratch_types=[
        pltpu.VMEM((2, C), jnp.int32),
        pltpu.VMEM((2, C), jnp.int32),
        pltpu.VMEM((C, HDIM), jnp.float32),
        pltpu.VMEM((C, HDIM), jnp.float32),
        pltpu.VMEM((C, HH), jnp.float32),
        pltpu.VMEM((C, HH), jnp.float32),
        pltpu.VMEM((N, HH), jnp.float32),
        pltpu.VMEM_SHARED((N, HDIM), jnp.float32),
        pltpu.SemaphoreType.DMA,
    ],
)


def _comb_body(p_ref, o_ref):
    o_ref[...] = jnp.concatenate([p_ref[0], p_ref[1]], axis=1)


def _combine(outpart):
    return pl.pallas_call(
        _comb_body,
        out_shape=jax.ShapeDtypeStruct((N, DIM), jnp.float32),
    )(outpart)


def kernel(x, edge_index, Wq, bq, Wk, bk, Wv, bv):
    qtab, ktab, vtab = _project(x, Wq, bq, Wk, bk, Wv, bv)
    zeros4 = jnp.zeros((N, HH), jnp.float32)
    zeros64 = jnp.zeros((N, HDIM), jnp.float32)
    score, sums = _scores_call(ktab, qtab, edge_index, zeros4)
    inv = _inv_sum(sums)
    outpart = _agg_call(vtab, edge_index, score, inv, zeros64)
    return _combine(outpart)


# speedup vs baseline: 2.1927x; 2.1927x over previous
import jax
import jax.numpy as jnp
from jax import lax
from jax.experimental import pallas as pl
from jax.experimental.pallas import tpu as pltpu
from jax.experimental.pallas import tpu_sc as plsc

N = 10000
E = 320000
DIM = 128
HDIM = 64
H = 8
HH = 4
HD = 16

C = 128
NCHUNKS = E // C
NCORES = 2
NSUB = 16
CPT = (NCHUNKS + NSUB - 1) // NSUB

_MESH = plsc.VectorSubcoreMesh(
    core_axis_name="c", subcore_axis_name="s",
    num_cores=NCORES, num_subcores=NSUB)

_SC_PARAMS = pltpu.CompilerParams(
    use_tc_tiling_on_sc=False, needs_layout_passes=False)


_PROJ_BLK = 1000


def _proj_body(x_ref, wq, bq, wk, bk, wv, bv, q_out, k_out, v_out):
    xb = x_ref[...]
    dn = (((1,), (1,)), ((), ()))
    q = lax.dot_general(xb, wq[...], dn,
                        preferred_element_type=jnp.float32) + bq[...]
    k = lax.dot_general(xb, wk[...], dn,
                        preferred_element_type=jnp.float32) + bk[...]
    v = lax.dot_general(xb, wv[...], dn,
                        preferred_element_type=jnp.float32) + bv[...]
    q_out[...] = jnp.stack([q[:, :HDIM], q[:, HDIM:]])
    k_out[...] = jnp.stack([k[:, :HDIM], k[:, HDIM:]])
    v_out[...] = jnp.stack([v[:, :HDIM], v[:, HDIM:]])


def _project(x, Wq, bq, Wk, bk, Wv, bv):
    full = pl.BlockSpec((DIM, DIM), lambda i: (0, 0))
    brow = pl.BlockSpec((1, DIM), lambda i: (0, 0))
    blk = pl.BlockSpec((_PROJ_BLK, DIM), lambda i: (i, 0))
    sblk = pl.BlockSpec((NCORES, _PROJ_BLK, HDIM), lambda i: (0, i, 0))
    ssds = jax.ShapeDtypeStruct((NCORES, N, HDIM), jnp.float32)
    return pl.pallas_call(
        _proj_body,
        grid=(N // _PROJ_BLK,),
        in_specs=[blk, full, brow, full, brow, full, brow],
        out_specs=[sblk] * 3,
        out_shape=[ssds] * 3,
    )(x, Wq, bq.reshape(1, DIM), Wk, bk.reshape(1, DIM), Wv, bv.reshape(1, DIM))


def _scores_body(ktab_hbm, qtab_hbm, ei_hbm, zero4_hbm,
                 score_hbm, sum_hbm,
                 ei_a, ei_b, kr_a, kr_b, qr_a, qr_b, scores_v, sum_acc,
                 sem):
    cid = lax.axis_index("c")
    sid = lax.axis_index("s")

    EI = (ei_a, ei_b)
    KR = (kr_a, kr_b)
    QR = (qr_a, qr_b)

    @pl.when(sid == 0)
    def _zero():
        pltpu.sync_copy(zero4_hbm, sum_acc)

    plsc.subcore_barrier()

    lanes = lax.iota(jnp.int32, HD)

    def load_idx(chunk, p):
        pltpu.sync_copy(ei_hbm.at[:, pl.ds(chunk * C, C)], EI[p])

    def issue_gathers(p):
        dk = pltpu.async_copy(ktab_hbm.at[cid].at[EI[p].at[0]], KR[p], sem)
        dq = pltpu.async_copy(qtab_hbm.at[cid].at[EI[p].at[1]], QR[p], sem)
        return dk, dq

    row_off = lanes >> 2
    col_idx = lanes & (HH - 1)
    lane_masks = [lanes == j for j in range(HD)]

    def compute(p):
        krows, qrows = KR[p], QR[p]

        def quad_body(qd, c2):
            e0 = qd * 4
            res = jnp.zeros((HD,), jnp.float32)
            for ej in range(4):
                for h in range(HH):
                    kv = krows[e0 + ej, pl.ds(h * HD, HD)]
                    qv = qrows[e0 + ej, pl.ds(h * HD, HD)]
                    tot = jnp.sum(kv * qv)
                    res = jnp.where(lane_masks[ej * HH + h],
                                    jnp.full((HD,), tot, jnp.float32),
                                    res)
            plsc.store_scatter(scores_v.at[pl.ds(e0, 4)],
                               [row_off, col_idx], res)
            return c2

        lax.fori_loop(0, C // 4, quad_body, 0)

    def do_writes(chunk, p):
        base = chunk * C
        pltpu.sync_copy(scores_v, score_hbm.at[cid, pl.ds(base, C)])
        pltpu.sync_copy(scores_v, sum_acc.at[EI[p].at[1]], add=True)

    load_idx(sid, 0)
    dk0, dq0 = issue_gathers(0)
    dk0.wait()
    dq0.wait()

    def dbl_body(i2, carry):
        for p in (0, 1):
            i = i2 * 2 + p
            cur = i * NSUB + sid
            nxt = cur + NSUB

            @pl.when(nxt < NCHUNKS)
            def _steady(p=p, cur=cur, nxt=nxt):
                load_idx(nxt, 1 - p)
                dk, dq = issue_gathers(1 - p)
                compute(p)
                do_writes(cur, p)
                dk.wait()
                dq.wait()

            @pl.when((cur < NCHUNKS) & (nxt >= NCHUNKS))
            def _tail(p=p, cur=cur):
                compute(p)
                do_writes(cur, p)
        return carry

    lax.fori_loop(0, (CPT + 1) // 2, dbl_body, 0)

    plsc.subcore_barrier()

    @pl.when(sid == 0)
    def _out():
        pltpu.sync_copy(sum_acc, sum_hbm.at[cid])


_scores_call = pl.kernel(
    _scores_body,
    out_type=(jax.ShapeDtypeStruct((NCORES, E, HH), jnp.float32),
              jax.ShapeDtypeStruct((NCORES, N, HH), jnp.float32)),
    mesh=_MESH,
    compiler_params=_SC_PARAMS,
    scratch_types=[
        pltpu.VMEM((2, C), jnp.int32),
        pltpu.VMEM((2, C), jnp.int32),
        pltpu.VMEM((C, HDIM), jnp.float32),
        pltpu.VMEM((C, HDIM), jnp.float32),
        pltpu.VMEM((C, HDIM), jnp.float32),
        pltpu.VMEM((C, HDIM), jnp.float32),
        pltpu.VMEM((C, HH), jnp.float32),
        pltpu.VMEM_SHARED((N, HH), jnp.float32),
        pltpu.SemaphoreType.DMA,
    ],
)


def _inv_body(p_ref, inv_ref):
    inv_ref[...] = 1.0 / p_ref[...]


def _inv_sum(sums):
    return pl.pallas_call(
        _inv_body,
        out_shape=jax.ShapeDtypeStruct((NCORES, N, HH), jnp.float32),
    )(sums)


def _agg_body(vtab_hbm, ei_hbm, score_hbm, inv_hbm, zero64_hbm,
              outpart_hbm,
              ei_a, ei_b, vr_a, vr_b, sc_a, sc_b, inv_t, out_acc,
              sem):
    cid = lax.axis_index("c")
    sid = lax.axis_index("s")

    EI = (ei_a, ei_b)
    VR = (vr_a, vr_b)
    SCV = (sc_a, sc_b)

    @pl.when(sid == 0)
    def _zero():
        pltpu.sync_copy(zero64_hbm, out_acc)

    pltpu.sync_copy(inv_hbm.at[cid], inv_t)
    plsc.subcore_barrier()

    lanes = lax.iota(jnp.int32, HD)
    row_off = lanes >> 2
    col_idx = lanes & (HH - 1)

    def load_idx(chunk, p):
        pltpu.sync_copy(ei_hbm.at[:, pl.ds(chunk * C, C)], EI[p])

    def issue_gathers(chunk, p):
        dv = pltpu.async_copy(vtab_hbm.at[cid].at[EI[p].at[0]], VR[p], sem)
        dscore = pltpu.async_copy(score_hbm.at[cid, pl.ds(chunk * C, C)],
                                  SCV[p], sem)
        return dv, dscore

    def compute(p):
        vrows, scores_v = VR[p], SCV[p]
        ei = EI[p]

        def group_body(g, c2):
            dvec = ei[1, pl.ds(g * HD, HD)]
            for jq in range(HD // 4):
                e0 = g * HD + 4 * jq
                rowi = jnp.zeros((HD,), jnp.int32)
                for ej in range(4):
                    mask = (lanes >= ej * HH) & (lanes < (ej + 1) * HH)
                    rowi = jnp.where(
                        mask,
                        jnp.full((HD,), dvec[4 * jq + ej], jnp.int32),
                        rowi)
                rowv = jnp.full((HD,), e0, jnp.int32) + row_off
                sv = plsc.load_gather(scores_v, [rowv, col_idx])
                iv = plsc.load_gather(inv_t, [rowi, col_idx])
                w = sv * iv
                for ej in range(4):
                    for h in range(HH):
                        ws = jnp.full((HD,), w[ej * HH + h], jnp.float32)
                        vrows[e0 + ej, pl.ds(h * HD, HD)] = (
                            vrows[e0 + ej, pl.ds(h * HD, HD)] * ws)
            return c2

        lax.fori_loop(0, C // HD, group_body, 0)

    def do_writes(p):
        pltpu.sync_copy(VR[p], out_acc.at[EI[p].at[1]], add=True)

    def chunk_body(i, carry):
        cur = i * NSUB + sid

        @pl.when(cur < NCHUNKS)
        def _():
            load_idx(cur, 0)
            dv, dscore = issue_gathers(cur, 0)
            dv.wait()
            dscore.wait()
            compute(0)
            do_writes(0)

        return carry

    lax.fori_loop(0, CPT, chunk_body, 0)

    plsc.subcore_barrier()

    @pl.when(sid == 0)
    def _out():
        pltpu.sync_copy(out_acc, outpart_hbm.at[cid])


_agg_call = pl.kernel(
    _agg_body,
    out_type=jax.ShapeDtypeStruct((NCORES, N, HDIM), jnp.float32),
    mesh=_MESH,
    compiler_params=_SC_PARAMS,
    scratch_types=[
        pltpu.VMEM((2, C), jnp.int32),
        pltpu.VMEM((2, C), jnp.int32),
        pltpu.VMEM((C, HDIM), jnp.float32),
        pltpu.VMEM((C, HDIM), jnp.float32),
        pltpu.VMEM((C, HH), jnp.float32),
        pltpu.VMEM((C, HH), jnp.float32),
        pltpu.VMEM((N, HH), jnp.float32),
        pltpu.VMEM_SHARED((N, HDIM), jnp.float32),
        pltpu.SemaphoreType.DMA,
    ],
)


def _comb_body(p_ref, o_ref):
    o_ref[...] = jnp.concatenate([p_ref[0], p_ref[1]], axis=1)


def _combine(outpart):
    return pl.pallas_call(
        _comb_body,
        out_shape=jax.ShapeDtypeStruct((N, DIM), jnp.float32),
    )(outpart)


def kernel(x, edge_index, Wq, bq, Wk, bk, Wv, bv):
    qtab, ktab, vtab = _project(x, Wq, bq, Wk, bk, Wv, bv)
    zeros4 = jnp.zeros((N, HH), jnp.float32)
    zeros64 = jnp.zeros((N, HDIM), jnp.float32)
    score, sums = _scores_call(ktab, qtab, edge_index, zeros4)
    inv = _inv_sum(sums)
    outpart = _agg_call(vtab, edge_index, score, inv, zeros64)
    return _combine(outpart)


# speedup vs baseline: 2.1939x; 1.0006x over previous
import functools

import jax
import jax.numpy as jnp
from jax import lax
from jax.experimental import pallas as pl
from jax.experimental.pallas import tpu as pltpu
from jax.experimental.pallas import tpu_sc as plsc

N = 10000
E = 320000
DIM = 128
HDIM = 64
H = 8
HH = 4
HD = 16

C = 128
NCHUNKS = E // C
NCORES = 2
NSUB = 16
CPT = (NCHUNKS + NSUB - 1) // NSUB

_SC_PARAMS = pltpu.CompilerParams(
    use_tc_tiling_on_sc=False, needs_layout_passes=False)


_PROJ_BLK = 1000


def _proj_body(x_ref, wq, bq, wk, bk, wv, bv, q_out, k_out, v_out):
    xb = x_ref[...]
    dn = (((1,), (1,)), ((), ()))
    q = lax.dot_general(xb, wq[...], dn,
                        preferred_element_type=jnp.float32) + bq[...]
    k = lax.dot_general(xb, wk[...], dn,
                        preferred_element_type=jnp.float32) + bk[...]
    v = lax.dot_general(xb, wv[...], dn,
                        preferred_element_type=jnp.float32) + bv[...]
    q_out[...] = jnp.stack([q[:, :HDIM], q[:, HDIM:]])
    k_out[...] = jnp.stack([k[:, :HDIM], k[:, HDIM:]])
    v_out[...] = jnp.stack([v[:, :HDIM], v[:, HDIM:]])


def _project(x, Wq, bq, Wk, bk, Wv, bv):
    full = pl.BlockSpec((DIM, DIM), lambda i: (0, 0))
    brow = pl.BlockSpec((1, DIM), lambda i: (0, 0))
    blk = pl.BlockSpec((_PROJ_BLK, DIM), lambda i: (i, 0))
    sblk = pl.BlockSpec((NCORES, _PROJ_BLK, HDIM), lambda i: (0, i, 0))
    ssds = jax.ShapeDtypeStruct((NCORES, N, HDIM), jnp.float32)
    return pl.pallas_call(
        _proj_body,
        grid=(N // _PROJ_BLK,),
        in_specs=[blk, full, brow, full, brow, full, brow],
        out_specs=[sblk] * 3,
        out_shape=[ssds] * 3,
    )(x, Wq, bq.reshape(1, DIM), Wk, bk.reshape(1, DIM), Wv, bv.reshape(1, DIM))


def _scores_body(ktab_hbm, qtab_hbm, ei_hbm, zero4_hbm,
                 score_hbm, sum_hbm,
                 ei_a, ei_b, kr_a, kr_b, qr_a, qr_b, scores_v, sum_acc,
                 sem):
    cid = lax.axis_index("c")
    sid = lax.axis_index("s")

    EI = (ei_a, ei_b)
    KR = (kr_a, kr_b)
    QR = (qr_a, qr_b)

    @pl.when(sid == 0)
    def _zero():
        pltpu.sync_copy(zero4_hbm, sum_acc)

    plsc.subcore_barrier()

    lanes = lax.iota(jnp.int32, HD)

    def load_idx(chunk, p):
        pltpu.sync_copy(ei_hbm.at[:, pl.ds(chunk * C, C)], EI[p])

    def issue_gathers(p):
        dk = pltpu.async_copy(ktab_hbm.at[cid].at[EI[p].at[0]], KR[p], sem)
        dq = pltpu.async_copy(qtab_hbm.at[cid].at[EI[p].at[1]], QR[p], sem)
        return dk, dq

    row_off = lanes >> 2
    col_idx = lanes & (HH - 1)
    lane_masks = [lanes == j for j in range(HD)]

    def compute(p):
        krows, qrows = KR[p], QR[p]

        def quad_body(qd, c2):
            e0 = qd * 4
            res = jnp.zeros((HD,), jnp.float32)
            for ej in range(4):
                for h in range(HH):
                    kv = krows[e0 + ej, pl.ds(h * HD, HD)]
                    qv = qrows[e0 + ej, pl.ds(h * HD, HD)]
                    tot = jnp.sum(kv * qv)
                    res = jnp.where(lane_masks[ej * HH + h],
                                    jnp.full((HD,), tot, jnp.float32),
                                    res)
            plsc.store_scatter(scores_v.at[pl.ds(e0, 4)],
                               [row_off, col_idx], res)
            return c2

        lax.fori_loop(0, C // 4, quad_body, 0)

    def do_writes(chunk, p):
        base = chunk * C
        pltpu.sync_copy(scores_v, score_hbm.at[cid, pl.ds(base, C)])
        pltpu.sync_copy(scores_v, sum_acc.at[EI[p].at[1]], add=True)

    load_idx(sid, 0)
    dk0, dq0 = issue_gathers(0)
    dk0.wait()
    dq0.wait()

    def dbl_body(i2, carry):
        for p in (0, 1):
            i = i2 * 2 + p
            cur = i * NSUB + sid
            nxt = cur + NSUB

            @pl.when(nxt < NCHUNKS)
            def _steady(p=p, cur=cur, nxt=nxt):
                load_idx(nxt, 1 - p)
                dk, dq = issue_gathers(1 - p)
                compute(p)
                do_writes(cur, p)
                dk.wait()
                dq.wait()

            @pl.when((cur < NCHUNKS) & (nxt >= NCHUNKS))
            def _tail(p=p, cur=cur):
                compute(p)
                do_writes(cur, p)
        return carry

    lax.fori_loop(0, (CPT + 1) // 2, dbl_body, 0)

    plsc.subcore_barrier()

    @pl.when(sid == 0)
    def _out():
        pltpu.sync_copy(sum_acc, sum_hbm.at[cid])


def _scores_call_build(mesh):
    return pl.kernel(
    _scores_body,
    out_type=(jax.ShapeDtypeStruct((NCORES, E, HH), jnp.float32),
              jax.ShapeDtypeStruct((NCORES, N, HH), jnp.float32)),
    mesh=mesh,
    compiler_params=_SC_PARAMS,
    scratch_types=[
        pltpu.VMEM((2, C), jnp.int32),
        pltpu.VMEM((2, C), jnp.int32),
        pltpu.VMEM((C, HDIM), jnp.float32),
        pltpu.VMEM((C, HDIM), jnp.float32),
        pltpu.VMEM((C, HDIM), jnp.float32),
        pltpu.VMEM((C, HDIM), jnp.float32),
        pltpu.VMEM((C, HH), jnp.float32),
        pltpu.VMEM_SHARED((N, HH), jnp.float32),
        pltpu.SemaphoreType.DMA,
    ],
    )


def _inv_body(p_ref, inv_ref):
    inv_ref[...] = 1.0 / p_ref[...]


def _inv_sum(sums):
    return pl.pallas_call(
        _inv_body,
        out_shape=jax.ShapeDtypeStruct((NCORES, N, HH), jnp.float32),
    )(sums)


def _agg_body(vtab_hbm, ei_hbm, score_hbm, inv_hbm, zero64_hbm,
              outpart_hbm,
              ei_a, ei_b, vr_a, vr_b, sc_a, sc_b, inv_t, out_acc,
              sem):
    cid = lax.axis_index("c")
    sid = lax.axis_index("s")

    EI = (ei_a, ei_b)
    VR = (vr_a, vr_b)
    SCV = (sc_a, sc_b)

    @pl.when(sid == 0)
    def _zero():
        pltpu.sync_copy(zero64_hbm, out_acc)

    pltpu.sync_copy(inv_hbm.at[cid], inv_t)
    plsc.subcore_barrier()

    lanes = lax.iota(jnp.int32, HD)
    row_off = lanes >> 2
    col_idx = lanes & (HH - 1)

    def load_idx(chunk, p):
        pltpu.sync_copy(ei_hbm.at[:, pl.ds(chunk * C, C)], EI[p])

    def issue_gathers(chunk, p):
        dv = pltpu.async_copy(vtab_hbm.at[cid].at[EI[p].at[0]], VR[p], sem)
        dscore = pltpu.async_copy(score_hbm.at[cid, pl.ds(chunk * C, C)],
                                  SCV[p], sem)
        return dv, dscore

    def compute(p):
        vrows, scores_v = VR[p], SCV[p]
        ei = EI[p]

        def group_body(g, c2):
            dvec = ei[1, pl.ds(g * HD, HD)]
            for jq in range(HD // 4):
                e0 = g * HD + 4 * jq
                rowi = jnp.zeros((HD,), jnp.int32)
                for ej in range(4):
                    mask = (lanes >= ej * HH) & (lanes < (ej + 1) * HH)
                    rowi = jnp.where(
                        mask,
                        jnp.full((HD,), dvec[4 * jq + ej], jnp.int32),
                        rowi)
                rowv = jnp.full((HD,), e0, jnp.int32) + row_off
                sv = plsc.load_gather(scores_v, [rowv, col_idx])
                iv = plsc.load_gather(inv_t, [rowi, col_idx])
                w = sv * iv
                for ej in range(4):
                    for h in range(HH):
                        ws = jnp.full((HD,), w[ej * HH + h], jnp.float32)
                        vrows[e0 + ej, pl.ds(h * HD, HD)] = (
                            vrows[e0 + ej, pl.ds(h * HD, HD)] * ws)
            return c2

        lax.fori_loop(0, C // HD, group_body, 0)

    def do_writes(p):
        pltpu.sync_copy(VR[p], out_acc.at[EI[p].at[1]], add=True)

    def chunk_body(i, carry):
        cur = i * NSUB + sid

        @pl.when(cur < NCHUNKS)
        def _():
            load_idx(cur, 0)
            dv, dscore = issue_gathers(cur, 0)
            dv.wait()
            dscore.wait()
            compute(0)
            do_writes(0)

        return carry

    lax.fori_loop(0, CPT, chunk_body, 0)

    plsc.subcore_barrier()

    @pl.when(sid == 0)
    def _out():
        pltpu.sync_copy(out_acc, outpart_hbm.at[cid])


def _agg_call_build(mesh):
    return pl.kernel(
    _agg_body,
    out_type=jax.ShapeDtypeStruct((NCORES, N, HDIM), jnp.float32),
    mesh=mesh,
    compiler_params=_SC_PARAMS,
    scratch_types=[
        pltpu.VMEM((2, C), jnp.int32),
        pltpu.VMEM((2, C), jnp.int32),
        pltpu.VMEM((C, HDIM), jnp.float32),
        pltpu.VMEM((C, HDIM), jnp.float32),
        pltpu.VMEM((C, HH), jnp.float32),
        pltpu.VMEM((C, HH), jnp.float32),
        pltpu.VMEM((N, HH), jnp.float32),
        pltpu.VMEM_SHARED((N, HDIM), jnp.float32),
        pltpu.SemaphoreType.DMA,
    ],
    )


def _comb_body(p_ref, o_ref):
    o_ref[...] = jnp.concatenate([p_ref[0], p_ref[1]], axis=1)


def _combine(outpart):
    return pl.pallas_call(
        _comb_body,
        out_shape=jax.ShapeDtypeStruct((N, DIM), jnp.float32),
    )(outpart)


@functools.lru_cache(maxsize=None)
def _sc_calls():
    mesh = plsc.VectorSubcoreMesh(
        core_axis_name="c", subcore_axis_name="s",
        num_cores=NCORES, num_subcores=NSUB)
    return _scores_call_build(mesh), _agg_call_build(mesh)


def kernel(x, edge_index, Wq, bq, Wk, bk, Wv, bv):
    _scores_call, _agg_call = _sc_calls()
    qtab, ktab, vtab = _project(x, Wq, bq, Wk, bk, Wv, bv)
    zeros4 = jnp.zeros((N, HH), jnp.float32)
    zeros64 = jnp.zeros((N, HDIM), jnp.float32)
    score, sums = _scores_call(ktab, qtab, edge_index, zeros4)
    inv = _inv_sum(sums)
    outpart = _agg_call(vtab, edge_index, score, inv, zeros64)
    return _combine(outpart)
